# baseline (speedup 1.0000x reference)
import functools

import jax
import jax.numpy as jnp
from jax import lax
from jax.experimental import pallas as pl
from jax.experimental.pallas import tpu as pltpu
from jax.experimental.pallas import tpu_sc as plsc

B, D = 16384, 128
G, K = 12, 12
NC, NS, L = 2, 16, 16
NW = NC * NS
RPW = B // NW
BLK = L
NBLK = RPW // BLK

_GATHER_DNUMS = lax.GatherDimensionNumbers(
    offset_dims=(), collapsed_slice_dims=(0,), start_index_map=(0,))


def _lane_splat(vec, j):
    idx = jnp.full((L, 1), j, jnp.int32)
    return lax.gather(vec, idx, _GATHER_DNUMS, (1,),
                      mode=lax.GatherScatterMode.PROMISE_IN_BOUNDS)


_mesh = plsc.VectorSubcoreMesh(
    core_axis_name="c", subcore_axis_name="s", num_cores=NC, num_subcores=NS)


@functools.partial(
    pl.kernel,
    out_type=jax.ShapeDtypeStruct((B * G,), jnp.float32),
    mesh=_mesh,
    compiler_params=pltpu.CompilerParams(needs_layout_passes=False),
    scratch_types=[
        pltpu.VMEM((RPW * D,), jnp.float32),
        pltpu.VMEM((G * L,), jnp.int32),
        pltpu.VMEM((RPW * G,), jnp.float32),
    ],
)
def _agg(inp_hbm, idx_hbm, out_hbm, rows_v, idx_v, out_v):
    wid = lax.axis_index("s") * NC + lax.axis_index("c")
    base = wid * RPW

    pltpu.sync_copy(idx_hbm, idx_v)
    pltpu.sync_copy(inp_hbm.at[pl.ds(base * D, RPW * D)], rows_v)

    iota = lax.broadcasted_iota(jnp.int32, (L,), 0)
    row_off = iota * D
    out_off = iota * G

    idx_rows = [idx_v[pl.ds(g * L, L)] for g in range(G)]

    def block_body(b, _):
        blk_off = row_off + b * (BLK * D)
        maxes = []
        for g in range(G):
            m = None
            for j in range(K):
                col = _lane_splat(idx_rows[g], j)
                v = plsc.load_gather(rows_v, [blk_off + col])
                m = v if m is None else jnp.maximum(m, v)
            maxes.append(m)

        mx = functools.reduce(jnp.maximum, maxes)
        exps = [jnp.exp(m - mx) for m in maxes]
        total = functools.reduce(lambda a, c: a + c, exps)
        inv = 1.0 / total

        blk_out_off = out_off + b * (BLK * G)
        for g in range(G):
            plsc.store_scatter(out_v, [blk_out_off + g], exps[g] * inv)
        return 0

    lax.fori_loop(0, NBLK, block_body, 0)

    pltpu.sync_copy(out_v, out_hbm.at[pl.ds(base * G, RPW * G)])


def kernel(inputs, subclass_indices):
    idx_pad = jnp.zeros((G, L), jnp.int32).at[:, :K].set(subclass_indices)
    out = _agg(inputs.reshape(B * D), idx_pad.reshape(G * L))
    return out.reshape(B, G)


# speedup vs baseline: 1.5363x; 1.5363x over previous
import functools

import jax
import jax.numpy as jnp
from jax import lax
from jax.experimental import pallas as pl
from jax.experimental.pallas import tpu as pltpu
from jax.experimental.pallas import tpu_sc as plsc

B, D = 16384, 128
G, K = 12, 12
NC, NS, L = 2, 16, 16
NW = NC * NS
RPW = B // NW
BLK = L
NBLK = RPW // BLK

_GATHER_DNUMS = lax.GatherDimensionNumbers(
    offset_dims=(), collapsed_slice_dims=(0,), start_index_map=(0,))


def _vperm(vec, perm):
    return lax.gather(vec, perm.reshape(L, 1), _GATHER_DNUMS, (1,),
                      mode=lax.GatherScatterMode.PROMISE_IN_BOUNDS)


_mesh = plsc.VectorSubcoreMesh(
    core_axis_name="c", subcore_axis_name="s", num_cores=NC, num_subcores=NS)


@functools.partial(
    pl.kernel,
    out_type=jax.ShapeDtypeStruct((B, G), jnp.float32),
    mesh=_mesh,
    compiler_params=pltpu.CompilerParams(
        needs_layout_passes=False, use_tc_tiling_on_sc=False,
        disable_bounds_checks=True),
    scratch_types=[
        pltpu.VMEM((RPW * D,), jnp.float32),
        pltpu.VMEM((G * L,), jnp.int32),
        pltpu.VMEM((RPW, G), jnp.float32),
    ],
)
def _agg(inp_hbm, idx_hbm, out_hbm, rows_flat, idx_v, out_v):
    wid = lax.axis_index("s") * NC + lax.axis_index("c")
    base = wid * RPW

    pltpu.sync_copy(idx_hbm, idx_v)
    pltpu.sync_copy(inp_hbm.at[pl.ds(base * D, RPW * D)], rows_flat)

    iota = lax.broadcasted_iota(jnp.int32, (L,), 0)
    row_off = iota * D
    rots = [((iota + j) % K).astype(jnp.int32) for j in range(K)]

    idx_rows = [idx_v[pl.ds(g * L, L)] for g in range(G)]

    def block_body(b, _):
        blk_base = row_off + b * (BLK * D)
        maxes = []
        for g in range(G):
            m = None
            for j in range(K):
                cols = _vperm(idx_rows[g], rots[j])
                v = plsc.load_gather(rows_flat, [blk_base + cols])
                m = v if m is None else jnp.maximum(m, v)
            maxes.append(m)

        mx = functools.reduce(jnp.maximum, maxes)
        exps = [jnp.exp(m - mx) for m in maxes]
        total = functools.reduce(lambda a, c: a + c, exps)
        inv = 1.0 / total

        row_ids = iota + b * BLK
        for g in range(G):
            plsc.store_scatter(out_v, [row_ids, jnp.full((L,), g, jnp.int32)],
                               exps[g] * inv)
        return 0

    lax.fori_loop(0, NBLK, block_body, 0)

    pltpu.sync_copy(out_v, out_hbm.at[pl.ds(base, RPW), :])


def kernel(inputs, subclass_indices):
    idx_pad = jnp.pad(subclass_indices, ((0, 0), (0, L - K)), mode="edge")
    return _agg(inputs.reshape(B * D), idx_pad.reshape(G * L))


# speedup vs baseline: 1.9457x; 1.2665x over previous
import functools

import jax
import jax.numpy as jnp
from jax import lax
from jax.experimental import pallas as pl
from jax.experimental.pallas import tpu as pltpu
from jax.experimental.pallas import tpu_sc as plsc

B, D = 16384, 128
G, K = 12, 12
NC, NS, L = 2, 16, 16
NW = NC * NS
RPW = B // NW
BLK = L
NBLK = RPW // BLK

_GATHER_DNUMS = lax.GatherDimensionNumbers(
    offset_dims=(), collapsed_slice_dims=(0,), start_index_map=(0,))


def _vperm(vec, perm):
    return lax.gather(vec, perm.reshape(L, 1), _GATHER_DNUMS, (1,),
                      mode=lax.GatherScatterMode.PROMISE_IN_BOUNDS)


_mesh = plsc.VectorSubcoreMesh(
    core_axis_name="c", subcore_axis_name="s", num_cores=NC, num_subcores=NS)


@functools.partial(
    pl.kernel,
    out_type=jax.ShapeDtypeStruct((G, B), jnp.float32),
    mesh=_mesh,
    compiler_params=pltpu.CompilerParams(
        needs_layout_passes=False, use_tc_tiling_on_sc=False,
        disable_bounds_checks=True),
    scratch_types=[
        pltpu.VMEM((RPW * D,), jnp.float32),
        pltpu.VMEM((G * L,), jnp.int32),
        pltpu.VMEM((G, RPW), jnp.float32),
    ],
)
def _agg(inp_hbm, idx_hbm, out_hbm, rows_flat, idx_v, out_v):
    wid = lax.axis_index("s") * NC + lax.axis_index("c")
    base = wid * RPW

    pltpu.sync_copy(idx_hbm, idx_v)
    pltpu.sync_copy(inp_hbm.at[pl.ds(base * D, RPW * D)], rows_flat)

    iota = lax.broadcasted_iota(jnp.int32, (L,), 0)
    row_off = iota * D
    rots = [((iota + j) % K).astype(jnp.int32) for j in range(K)]

    idx_rows = [idx_v[pl.ds(g * L, L)] for g in range(G)]

    def block_body(b, _):
        blk_base = row_off + b * (BLK * D)
        maxes = []
        for g in range(G):
            m = None
            for j in range(K):
                cols = _vperm(idx_rows[g], rots[j])
                v = plsc.load_gather(rows_flat, [blk_base + cols])
                m = v if m is None else jnp.maximum(m, v)
            maxes.append(m)

        mx = functools.reduce(jnp.maximum, maxes)
        exps = [jnp.exp(m - mx) for m in maxes]
        total = functools.reduce(lambda a, c: a + c, exps)
        inv = 1.0 / total

        for g in range(G):
            out_v[g, pl.ds(b * BLK, BLK)] = exps[g] * inv
        return 0

    lax.fori_loop(0, NBLK, block_body, 0)

    pltpu.sync_copy(out_v, out_hbm.at[:, pl.ds(base, RPW)])


def kernel(inputs, subclass_indices):
    idx_pad = jnp.pad(subclass_indices, ((0, 0), (0, L - K)))
    return _agg(inputs.reshape(B * D), idx_pad.reshape(G * L)).T
